# baseline (speedup 1.0000x reference)
import jax
import jax.numpy as jnp
import numpy as np
from jax import lax
from jax.experimental import pallas as pl
from jax.experimental.pallas import tpu as pltpu
from jax.experimental.pallas import tpu_sc as plsc

_VOCAB = 1000000
_EMB = 64
_BATCH = 16
_SEQ = 2048

_NC, _NS, _L = 2, 16, 16
_NW = _NC * _NS
_PER_W = _BATCH * _SEQ // _NW
_C = 128
_NCHUNK = _PER_W // _C
_W = 4096


def _pos_table(emb, seq):
    enc = np.zeros((seq, emb), dtype=np.float32)
    pos = np.arange(0.0, seq, dtype=np.float32)[:, None]
    i2 = np.arange(0, emb, 2).astype(np.float32)
    enc[:, 0::2] = np.sin(pos / 10000 ** (i2 / emb))
    enc[:, 1::2] = np.cos(pos / 10000 ** (i2 / emb))
    return enc


_P = _pos_table(_EMB, _SEQ)
_POS_EO = np.ascontiguousarray(
    np.concatenate([_P[:, 0::2], _P[:, 1::2]], axis=1)
)


def _pack(table):
    u = lax.bitcast_convert_type(table, jnp.int32)
    pe = lax.shift_right_logical(u[:, 0::2] + jnp.int32(0x8000), 16)
    po = (u[:, 1::2] + jnp.int32(0x8000)) & jnp.int32(-65536)
    return pe | po


def _sc_body(tbl_hbm, x_hbm, pos_hbm, out_hbm, idx2, rows2, posv2, rowsf2,
             sem_r0, sem_r1, sem_s0, sem_s1):
    wid = lax.axis_index("s") * _NC + lax.axis_index("c")
    b = wid // 2
    t_half = (wid % 2) * _PER_W
    sem_r = (sem_r0, sem_r1)
    sem_s = (sem_s0, sem_s1)

    def load_and_fire(c):
        p = c % 2
        t = t_half + c * _C
        pltpu.sync_copy(x_hbm.at[b, pl.ds(t, _C)], idx2.at[p])
        pltpu.async_copy(pos_hbm.at[pl.ds(t, _C), :], posv2.at[p], sem_r[p])

        @pl.loop(0, _C // _L)
        def _fire(g):
            rv = idx2[p, pl.ds(g * _L, _L)]
            for j in range(_L):
                pltpu.async_copy(
                    tbl_hbm.at[rv[j], :], rows2.at[p, g * _L + j], sem_r[p]
                )

    load_and_fire(0)
    for c in range(_NCHUNK):
        p = c % 2
        t = t_half + c * _C
        if c + 1 < _NCHUNK:
            if c >= 1:
                pltpu.make_async_copy(
                    rowsf2.at[1 - p], out_hbm.at[b, pl.ds(t, _C), :],
                    sem_s[1 - p],
                ).wait()
            load_and_fire(c + 1)

        pltpu.make_async_copy(
            pos_hbm.at[pl.ds(t, _C), :], posv2.at[p], sem_r[p]
        ).wait()
        pltpu.make_async_copy(
            tbl_hbm.at[pl.ds(0, _C), :], rows2.at[p], sem_r[p]
        ).wait()

        @pl.loop(0, _C)
        def _cv(i):
            for k in range(2):
                u = rows2[p, i, pl.ds(k * _L, _L)]
                lo = plsc.bitcast(lax.shift_left(u, 16), jnp.float32)
                hi = plsc.bitcast(u & jnp.int32(-65536), jnp.float32)
                lo = lo + posv2[p, i, pl.ds(k * _L, _L)]
                hi = hi + posv2[p, i, pl.ds(32 + k * _L, _L)]
                ev = lax.iota(jnp.int32, _L) * 2 + (k * 2 * _L)
                plsc.store_scatter(rowsf2.at[p, i], [ev], lo)
                plsc.store_scatter(rowsf2.at[p, i], [ev + 1], hi)

        pltpu.async_copy(
            rowsf2.at[p], out_hbm.at[b, pl.ds(t, _C), :], sem_s[p]
        )

    pltpu.make_async_copy(
        rowsf2.at[0], out_hbm.at[b, pl.ds(t_half, _C), :], sem_s[0]
    ).wait()
    pltpu.make_async_copy(
        rowsf2.at[1], out_hbm.at[b, pl.ds(t_half, _C), :], sem_s[1]
    ).wait()


@jax.jit
def _pos_ntok(x, table):
    packed = _pack(table)
    mesh = plsc.VectorSubcoreMesh(core_axis_name="c", subcore_axis_name="s")
    fn = pl.kernel(
        _sc_body,
        out_type=jax.ShapeDtypeStruct((_BATCH, _SEQ, _EMB), jnp.float32),
        mesh=mesh,
        scratch_types=[
            pltpu.VMEM((2, _C), jnp.int32),
            pltpu.VMEM((2, _C, _EMB // 2), jnp.int32),
            pltpu.VMEM((2, _C, _EMB), jnp.float32),
            pltpu.VMEM((2, _C, _EMB), jnp.float32),
            pltpu.SemaphoreType.DMA,
            pltpu.SemaphoreType.DMA,
            pltpu.SemaphoreType.DMA,
            pltpu.SemaphoreType.DMA,
        ],
        compiler_params=pltpu.CompilerParams(needs_layout_passes=False),
    )
    return fn(packed, x, jnp.asarray(_POS_EO))


def kernel(x, table):
    return _pos_ntok(x, table)


# speedup vs baseline: 25.4128x; 25.4128x over previous
import jax
import jax.numpy as jnp
import numpy as np
from jax import lax
from jax.experimental import pallas as pl
from jax.experimental.pallas import tpu as pltpu
from jax.experimental.pallas import tpu_sc as plsc

_VOCAB = 1000000
_EMB = 64
_BATCH = 16
_SEQ = 2048

_NC, _NS, _L = 2, 16, 16
_NW = _NC * _NS
_PER_W = _BATCH * _SEQ // _NW
_C = 128
_NCHUNK = _PER_W // _C


def _pos_table(emb, seq):
    enc = np.zeros((seq, emb), dtype=np.float32)
    pos = np.arange(0.0, seq, dtype=np.float32)[:, None]
    i2 = np.arange(0, emb, 2).astype(np.float32)
    enc[:, 0::2] = np.sin(pos / 10000 ** (i2 / emb))
    enc[:, 1::2] = np.cos(pos / 10000 ** (i2 / emb))
    return enc


_POS = _pos_table(_EMB, _SEQ)


def _sc_body(table_hbm, x_hbm, pos_hbm, out_hbm, idx2, rows2, pos2,
             sem_r0, sem_r1, sem_s0, sem_s1):
    wid = lax.axis_index("s") * _NC + lax.axis_index("c")
    b = wid // 2
    t_half = (wid % 2) * _PER_W
    sem_r = (sem_r0, sem_r1)
    sem_s = (sem_s0, sem_s1)

    def load_and_fire(c):
        p = c % 2
        t = t_half + c * _C
        pltpu.sync_copy(x_hbm.at[b, pl.ds(t, _C)], idx2.at[p])
        pltpu.async_copy(pos_hbm.at[pl.ds(t, _C), :], pos2.at[p], sem_r[p])

        @pl.loop(0, _C // _L)
        def _fire(g):
            rv = idx2[p, pl.ds(g * _L, _L)]
            for j in range(_L):
                pltpu.async_copy(
                    table_hbm.at[rv[j], :], rows2.at[p, g * _L + j], sem_r[p]
                )

    load_and_fire(0)
    for c in range(_NCHUNK):
        p = c % 2
        t = t_half + c * _C
        if c + 1 < _NCHUNK:
            if c >= 1:
                pltpu.make_async_copy(
                    rows2.at[1 - p], out_hbm.at[b, pl.ds(t, _C), :],
                    sem_s[1 - p],
                ).wait()
            load_and_fire(c + 1)

        pltpu.make_async_copy(
            pos_hbm.at[pl.ds(t, _C), :], pos2.at[p], sem_r[p]
        ).wait()
        pltpu.make_async_copy(
            table_hbm.at[pl.ds(0, _C), :], rows2.at[p], sem_r[p]
        ).wait()

        @pl.loop(0, _C)
        def _add(i):
            for k in range(_EMB // _L):
                sl = pl.ds(k * _L, _L)
                rows2[p, i, sl] = rows2[p, i, sl] + pos2[p, i, sl]

        pltpu.async_copy(rows2.at[p], out_hbm.at[b, pl.ds(t, _C), :], sem_s[p])

    pltpu.make_async_copy(
        rows2.at[0], out_hbm.at[b, pl.ds(t_half, _C), :], sem_s[0]
    ).wait()
    pltpu.make_async_copy(
        rows2.at[1], out_hbm.at[b, pl.ds(t_half, _C), :], sem_s[1]
    ).wait()


@jax.jit
def _pos_ntok(x, table):
    mesh = plsc.VectorSubcoreMesh(core_axis_name="c", subcore_axis_name="s")
    fn = pl.kernel(
        _sc_body,
        out_type=jax.ShapeDtypeStruct((_BATCH, _SEQ, _EMB), jnp.float32),
        mesh=mesh,
        scratch_types=[
            pltpu.VMEM((2, _C), jnp.int32),
            pltpu.VMEM((2, _C, _EMB), jnp.float32),
            pltpu.VMEM((2, _C, _EMB), jnp.float32),
            pltpu.SemaphoreType.DMA,
            pltpu.SemaphoreType.DMA,
            pltpu.SemaphoreType.DMA,
            pltpu.SemaphoreType.DMA,
        ],
    )
    return fn(table, x, jnp.asarray(_POS))


def kernel(x, table):
    return _pos_ntok(x, table)


# speedup vs baseline: 25.4285x; 1.0006x over previous
import jax
import jax.numpy as jnp
import numpy as np
from jax import lax
from jax.experimental import pallas as pl
from jax.experimental.pallas import tpu as pltpu
from jax.experimental.pallas import tpu_sc as plsc

_VOCAB = 1000000
_EMB = 64
_BATCH = 16
_SEQ = 2048

_NC, _NS, _L = 2, 16, 16
_NW = _NC * _NS
_PER_W = _BATCH * _SEQ // _NW
_C = 128
_NCHUNK = _PER_W // _C
_W = 4096


def _pos_table(emb, seq):
    enc = np.zeros((seq, emb), dtype=np.float32)
    pos = np.arange(0.0, seq, dtype=np.float32)[:, None]
    i2 = np.arange(0, emb, 2).astype(np.float32)
    enc[:, 0::2] = np.sin(pos / 10000 ** (i2 / emb))
    enc[:, 1::2] = np.cos(pos / 10000 ** (i2 / emb))
    return enc


_POS = _pos_table(_EMB, _SEQ)


def _tc_pack_body(x_ref, o_ref):
    u = lax.bitcast_convert_type(x_ref[...], jnp.int32)
    lo = lax.shift_right_logical(u[: _EMB // 2, :] + jnp.int32(0x8000), 16)
    hi = (u[_EMB // 2 :, :] + jnp.int32(0x8000)) & jnp.int32(-65536)
    o_ref[...] = jnp.transpose(lo | hi, (1, 0))


def _tc_pack(table_t):
    return pl.pallas_call(
        _tc_pack_body,
        grid=((_VOCAB + _W - 1) // _W,),
        in_specs=[pl.BlockSpec((_EMB, _W), lambda i: (0, i))],
        out_specs=pl.BlockSpec((_W, _EMB // 2), lambda i: (i, 0)),
        out_shape=jax.ShapeDtypeStruct((_VOCAB, _EMB // 2), jnp.int32),
    )(table_t)


def _sc_body(tbl_hbm, x_hbm, pos_hbm, out_hbm, idx2, rows2, pos2, rowsf2,
             sem_r0, sem_r1, sem_s0, sem_s1):
    wid = lax.axis_index("s") * _NC + lax.axis_index("c")
    b = wid // 2
    t_half = (wid % 2) * _PER_W
    sem_r = (sem_r0, sem_r1)
    sem_s = (sem_s0, sem_s1)

    def load_and_fire(c):
        p = c % 2
        t = t_half + c * _C
        pltpu.sync_copy(x_hbm.at[b, pl.ds(t, _C)], idx2.at[p])
        pltpu.async_copy(pos_hbm.at[pl.ds(t, _C), :], pos2.at[p], sem_r[p])

        @pl.loop(0, _C // _L)
        def _fire(g):
            rv = idx2[p, pl.ds(g * _L, _L)]
            for j in range(_L):
                pltpu.async_copy(
                    tbl_hbm.at[rv[j], :], rows2.at[p, g * _L + j], sem_r[p]
                )

    load_and_fire(0)
    for c in range(_NCHUNK):
        p = c % 2
        t = t_half + c * _C
        if c + 1 < _NCHUNK:
            if c >= 1:
                pltpu.make_async_copy(
                    rowsf2.at[1 - p], out_hbm.at[b, pl.ds(t, _C), :],
                    sem_s[1 - p],
                ).wait()
            load_and_fire(c + 1)

        pltpu.make_async_copy(
            pos_hbm.at[pl.ds(t, _C), :], pos2.at[p], sem_r[p]
        ).wait()
        pltpu.make_async_copy(
            tbl_hbm.at[pl.ds(0, _C), :], rows2.at[p], sem_r[p]
        ).wait()

        @pl.loop(0, _C)
        def _cv(i):
            for k in range(2):
                sl = pl.ds(k * _L, _L)
                sh = pl.ds(_EMB // 2 + k * _L, _L)
                u = rows2[p, i, sl]
                lo = plsc.bitcast(lax.shift_left(u, 16), jnp.float32)
                hi = plsc.bitcast(u & jnp.int32(-65536), jnp.float32)
                rowsf2[p, i, sl] = lo + pos2[p, i, sl]
                rowsf2[p, i, sh] = hi + pos2[p, i, sh]

        pltpu.async_copy(
            rowsf2.at[p], out_hbm.at[b, pl.ds(t, _C), :], sem_s[p]
        )

    pltpu.make_async_copy(
        rowsf2.at[0], out_hbm.at[b, pl.ds(t_half, _C), :], sem_s[0]
    ).wait()
    pltpu.make_async_copy(
        rowsf2.at[1], out_hbm.at[b, pl.ds(t_half, _C), :], sem_s[1]
    ).wait()


@jax.jit
def _pos_ntok(x, table):
    packed = _tc_pack(table.T)
    mesh = plsc.VectorSubcoreMesh(core_axis_name="c", subcore_axis_name="s")
    fn = pl.kernel(
        _sc_body,
        out_type=jax.ShapeDtypeStruct((_BATCH, _SEQ, _EMB), jnp.float32),
        mesh=mesh,
        scratch_types=[
            pltpu.VMEM((2, _C), jnp.int32),
            pltpu.VMEM((2, _C, _EMB // 2), jnp.int32),
            pltpu.VMEM((2, _C, _EMB), jnp.float32),
            pltpu.VMEM((2, _C, _EMB), jnp.float32),
            pltpu.SemaphoreType.DMA,
            pltpu.SemaphoreType.DMA,
            pltpu.SemaphoreType.DMA,
            pltpu.SemaphoreType.DMA,
        ],
        compiler_params=pltpu.CompilerParams(needs_layout_passes=False),
    )
    return fn(packed, x, jnp.asarray(_POS))


def kernel(x, table):
    return _pos_ntok(x, table)


# speedup vs baseline: 25.4627x; 1.0013x over previous
import jax
import jax.numpy as jnp
import numpy as np
from jax import lax
from jax.experimental import pallas as pl
from jax.experimental.pallas import tpu as pltpu
from jax.experimental.pallas import tpu_sc as plsc

_VOCAB = 1000000
_EMB = 64
_BATCH = 16
_SEQ = 2048

_NC, _NS, _L = 2, 16, 16
_NW = _NC * _NS
_PER_W = _BATCH * _SEQ // _NW
_C = 128
_NCHUNK = _PER_W // _C


def _pos_table(emb, seq):
    enc = np.zeros((seq, emb), dtype=np.float32)
    pos = np.arange(0.0, seq, dtype=np.float32)[:, None]
    i2 = np.arange(0, emb, 2).astype(np.float32)
    enc[:, 0::2] = np.sin(pos / 10000 ** (i2 / emb))
    enc[:, 1::2] = np.cos(pos / 10000 ** (i2 / emb))
    return enc


_POS = _pos_table(_EMB, _SEQ)


def _sc_body(table_hbm, x_hbm, pos_hbm, out_hbm, idx2, rows2, pos2,
             sem_r0, sem_r1, sem_s0, sem_s1):
    wid = lax.axis_index("s") * _NC + lax.axis_index("c")
    b = wid // 2
    t_half = (wid % 2) * _PER_W
    sem_r = (sem_r0, sem_r1)
    sem_s = (sem_s0, sem_s1)

    def load_and_fire(c):
        p = c % 2
        t = t_half + c * _C
        pltpu.sync_copy(x_hbm.at[b, pl.ds(t, _C)], idx2.at[p])
        pltpu.async_copy(pos_hbm.at[pl.ds(t, _C), :], pos2.at[p], sem_r[p])

        @pl.loop(0, _C // _L)
        def _fire(g):
            rv = idx2[p, pl.ds(g * _L, _L)]
            for j in range(_L):
                pltpu.async_copy(
                    table_hbm.at[rv[j], :], rows2.at[p, g * _L + j], sem_r[p]
                )

    load_and_fire(0)
    for c in range(_NCHUNK):
        p = c % 2
        t = t_half + c * _C
        if c + 1 < _NCHUNK:
            if c >= 1:
                pltpu.make_async_copy(
                    rows2.at[1 - p], out_hbm.at[b, pl.ds(t, _C), :],
                    sem_s[1 - p],
                ).wait()
            load_and_fire(c + 1)

        pltpu.make_async_copy(
            pos_hbm.at[pl.ds(t, _C), :], pos2.at[p], sem_r[p]
        ).wait()
        pltpu.make_async_copy(
            table_hbm.at[pl.ds(0, _C), :], rows2.at[p], sem_r[p]
        ).wait()

        @pl.loop(0, _C)
        def _add(i):
            for k in range(_EMB // _L):
                sl = pl.ds(k * _L, _L)
                rows2[p, i, sl] = rows2[p, i, sl] + pos2[p, i, sl]

        pltpu.async_copy(rows2.at[p], out_hbm.at[b, pl.ds(t, _C), :], sem_s[p])

    pltpu.make_async_copy(
        rows2.at[0], out_hbm.at[b, pl.ds(t_half, _C), :], sem_s[0]
    ).wait()
    pltpu.make_async_copy(
        rows2.at[1], out_hbm.at[b, pl.ds(t_half, _C), :], sem_s[1]
    ).wait()


@jax.jit
def _pos_ntok(x, table):
    mesh = plsc.VectorSubcoreMesh(core_axis_name="c", subcore_axis_name="s")
    fn = pl.kernel(
        _sc_body,
        out_type=jax.ShapeDtypeStruct((_BATCH, _SEQ, _EMB), jnp.float32),
        mesh=mesh,
        scratch_types=[
            pltpu.VMEM((2, _C), jnp.int32),
            pltpu.VMEM((2, _C, _EMB), jnp.float32),
            pltpu.VMEM((2, _C, _EMB), jnp.float32),
            pltpu.SemaphoreType.DMA,
            pltpu.SemaphoreType.DMA,
            pltpu.SemaphoreType.DMA,
            pltpu.SemaphoreType.DMA,
        ],
    )
    return fn(table, x, jnp.asarray(_POS))


def kernel(x, table):
    return _pos_ntok(x, table)
